# baseline (speedup 1.0000x reference)
import functools

import jax
import jax.numpy as jnp
from jax import lax
from jax.experimental import pallas as pl
from jax.experimental.pallas import tpu as pltpu
from jax.experimental.pallas import tpu_sc as plsc

BATCH = 16384
EDIM = 64
ROW = 2 * EDIM
NC = 2
NS = 16
NW = NC * NS
CHUNK = 128
NCHUNKS = BATCH // CHUNK
NCH = NCHUNKS // NW
LANES = 16
NSLOT = 3
NG = EDIM // LANES

_f32 = jnp.float32


def _pack_body(umf_t, umlp_t, imf_t, imlp_t, wpm, out_u, out_i):
  out_u[:, :EDIM] = umf_t[...].T * wpm[...]
  out_u[:, EDIM:] = umlp_t[...].T
  out_i[:, :EDIM] = imf_t[...].T
  out_i[:, EDIM:] = imlp_t[...].T


def _pack_tables(user_mf, user_mlp, item_mf, item_mlp, wpm):
  v = user_mf.shape[0]
  tab = jax.ShapeDtypeStruct((v, ROW), _f32)
  return pl.pallas_call(
      _pack_body,
      out_shape=[tab, tab],
  )(user_mf.T, user_mlp.T, item_mf.T, item_mlp.T, wpm)


def _sc_gather_body(user_hbm, item_hbm, utab, itab, out_x, out_mfp,
                    idx_u, idx_i, buf_u, buf_i, mfp_v, sem, wsem):
  wid = lax.axis_index("s") * NC + lax.axis_index("c")
  r0 = wid * NCH
  pltpu.sync_copy(user_hbm.at[pl.ds(r0, NCH)], idx_u)
  pltpu.sync_copy(item_hbm.at[pl.ds(r0, NCH)], idx_i)
  i16 = lax.iota(jnp.int32, LANES)
  lane_hi = i16 // 8
  lane_lo = i16 % 8
  zero16 = i16 * 0

  def fire_gather(j):
    s = j % NSLOT
    return (pltpu.async_copy(utab.at[idx_u.at[j]], buf_u.at[s], sem),
            pltpu.async_copy(itab.at[idx_i.at[j]], buf_i.at[s], sem))

  def fire_writes(j):
    s = j % NSLOT
    return (pltpu.async_copy(buf_u.at[s, :, pl.ds(EDIM, EDIM)],
                             out_x.at[r0 + j, :, pl.ds(0, EDIM)], wsem),
            pltpu.async_copy(buf_i.at[s, :, pl.ds(EDIM, EDIM)],
                             out_x.at[r0 + j, :, pl.ds(EDIM, EDIM)], wsem))

  g = {}
  for j in range(min(NSLOT, NCH)):
    g[j] = fire_gather(j)
  w = {}
  for j in range(NCH):
    s = j % NSLOT
    g[j][0].wait()
    g[j][1].wait()
    nxt = NSLOT + j - 1
    if j >= 1 and nxt < NCH:
      w[j - 1][0].wait()
      w[j - 1][1].wait()
      g[nxt] = fire_gather(nxt)

    def mf_row(r, c):
      acc = (buf_u[s, r, pl.ds(0, LANES)] * buf_i[s, r, pl.ds(0, LANES)])
      for k in range(1, NG):
        acc += (buf_u[s, r, pl.ds(k * LANES, LANES)]
                * buf_i[s, r, pl.ds(k * LANES, LANES)])
      plsc.store_scatter(mfp_v, [i16 + j * LANES, zero16 + r], acc)
      return c

    lax.fori_loop(0, CHUNK, mf_row, 0, unroll=4)
    w[j] = fire_writes(j)
  tail = pltpu.async_copy(mfp_v, out_mfp.at[pl.ds(r0 * LANES, NCH * LANES)],
                          wsem)
  for j in range(max(0, NCH - NSLOT), NCH):
    w[j][0].wait()
    w[j][1].wait()
  tail.wait()


def _sc_gather(user2d, item2d, utab, itab):
  mesh = plsc.VectorSubcoreMesh(core_axis_name="c", subcore_axis_name="s")
  fn = functools.partial(
      pl.kernel,
      mesh=mesh,
      out_type=[jax.ShapeDtypeStruct((NCHUNKS, CHUNK, ROW), _f32),
                jax.ShapeDtypeStruct((NCHUNKS * LANES, CHUNK), _f32)],
      scratch_types=[
          pltpu.VMEM((NCH, CHUNK), jnp.int32),
          pltpu.VMEM((NCH, CHUNK), jnp.int32),
          pltpu.VMEM((NSLOT, CHUNK, ROW), _f32),
          pltpu.VMEM((NSLOT, CHUNK, ROW), _f32),
          pltpu.VMEM((NCH * LANES, CHUNK), _f32),
          pltpu.SemaphoreType.DMA,
          pltpu.SemaphoreType.DMA,
      ],
      compiler_params=pltpu.CompilerParams(use_tc_tiling_on_sc=False,
                                           needs_layout_passes=False),
  )(_sc_gather_body)
  return fn(user2d, item2d, utab, itab)


def _tc_body(xref, mfpref, w1, b1, w2, b2, wph, bp, out):
  x = xref[...]
  h1 = lax.dot_general(x, w1[...], (((1,), (1,)), ((), ())),
                       preferred_element_type=_f32)
  h1 = jnp.maximum(h1 + b1[...], 0.0)
  h2 = lax.dot_general(h1, w2[...], (((1,), (1,)), ((), ())),
                       preferred_element_type=_f32)
  h2 = jnp.maximum(h2 + b2[...], 0.0)
  s = jnp.sum(h2 * wph[...], axis=1, keepdims=True)
  m4 = mfpref[...]
  smf = jnp.sum(jnp.sum(m4, axis=1), axis=1)
  nrow = out.shape[1]
  logit = s.reshape(nrow, CHUNK) + smf + bp[...]
  out[...] = jax.nn.sigmoid(logit).reshape(out.shape)


def _tc_mlp(xrows, mfp, W1, b1, W2, b2, wph, bp):
  blk = 2048
  grid = BATCH // blk
  out2 = pl.pallas_call(
      _tc_body,
      grid=(grid,),
      in_specs=[pl.BlockSpec((blk, ROW), lambda i: (i, 0)),
                pl.BlockSpec((blk // CHUNK, 2, 8, CHUNK),
                             lambda i: (i, 0, 0, 0)),
                pl.BlockSpec((128, 128), lambda i: (0, 0)),
                pl.BlockSpec((1, 128), lambda i: (0, 0)),
                pl.BlockSpec((64, 128), lambda i: (0, 0)),
                pl.BlockSpec((1, 64), lambda i: (0, 0)),
                pl.BlockSpec((1, 64), lambda i: (0, 0)),
                pl.BlockSpec((1, 1), lambda i: (0, 0))],
      out_specs=pl.BlockSpec((1, blk // CHUNK, CHUNK), lambda i: (i, 0, 0)),
      out_shape=jax.ShapeDtypeStruct((grid, blk // CHUNK, CHUNK), _f32),
  )(xrows, mfp, W1, b1, W2, b2, wph, bp)
  return out2.reshape(BATCH)


def kernel(user, item, user_mf, item_mf, user_mlp, item_mlp,
           W1, b1, W2, b2, Wp, bp):
  user2d = user.astype(jnp.int32).reshape(NCHUNKS, CHUNK)
  item2d = item.astype(jnp.int32).reshape(NCHUNKS, CHUNK)
  wp = Wp.reshape(128)
  wpm = wp[:EDIM].reshape(1, EDIM)
  wph = wp[EDIM:].reshape(1, EDIM)
  utab, itab = _pack_tables(user_mf, user_mlp, item_mf, item_mlp, wpm)
  xrows3, mfp2 = _sc_gather(user2d, item2d, utab, itab)
  mfp4 = mfp2.reshape(NCHUNKS, 2, 8, CHUNK)
  xrows = xrows3.reshape(BATCH, ROW)
  return _tc_mlp(xrows, mfp4, W1, b1.reshape(1, 128), W2, b2.reshape(1, 64),
                 wph, bp.reshape(1, 1))
